# baseline (speedup 1.0000x reference)
import functools

import jax
import jax.numpy as jnp
from jax import lax
from jax.experimental import pallas as pl
from jax.experimental.pallas import tpu as pltpu
from jax.experimental.pallas import tpu_sc as plsc

USER = 5000
N_REAL = 10000
NROW = 10240
EMB = 256
HALF = 128
NE = 160000
NE_PAD = 163840
PAD_NODE = 10000
NC, NS = 2, 16
NW = NC * NS
EPT_DEG = NE_PAD // NW
CHUNK = 128
NCHUNK = NE_PAD // NS // CHUNK
ROWS_PER_TILE = NROW // NS
BATCH = 4096
LMBD = 1e-05
BLK = 1280
BB = 512

_MESH = plsc.VectorSubcoreMesh(
    core_axis_name="c", subcore_axis_name="s", num_cores=NC, num_subcores=NS)


def _deg_body(src_hbm, out_hbm, src_v, hist_v):
    c = lax.axis_index("c")
    s = lax.axis_index("s")
    w = s * NC + c
    pltpu.sync_copy(src_hbm.at[w], src_v)

    def zero(i, carry):
        hist_v[pl.ds(i * 16, 16)] = jnp.zeros((16,), jnp.float32)
        return carry
    lax.fori_loop(0, NROW // 16, zero, 0)

    ones = jnp.ones((16,), jnp.float32)

    def body(i, carry):
        idx = src_v[pl.ds(i * 16, 16)]
        plsc.addupdate_scatter(hist_v, [idx], ones)
        return carry
    lax.fori_loop(0, EPT_DEG // 16, body, 0)
    pltpu.sync_copy(hist_v, out_hbm.at[w])


_SC_PARAMS = pltpu.CompilerParams(needs_layout_passes=False)

_deg = pl.kernel(
    _deg_body,
    out_type=jax.ShapeDtypeStruct((NW, NROW), jnp.float32),
    mesh=_MESH,
    compiler_params=_SC_PARAMS,
    scratch_types=[
        pltpu.VMEM((EPT_DEG,), jnp.int32),
        pltpu.VMEM((NROW,), jnp.float32),
    ],
)


def _seg_body(ga_hbm, gb_hbm, src_hbm, dst_hbm, z_hbm, sa_hbm, sb_hbm,
              src_i, dst_ring, rows_v, sem0, sem1, dsem0, dsem1, acc_sp):
    c = lax.axis_index("c")
    s = lax.axis_index("s")
    pltpu.sync_copy(src_hbm.at[s], src_i)
    sl = pl.ds(s * ROWS_PER_TILE, ROWS_PER_TILE)
    pltpu.sync_copy(z_hbm, acc_sp.at[sl])
    plsc.subcore_barrier()

    def run(g_hbm):
        sems = (sem0, sem1)
        dsems = (dsem0, dsem1)
        for b in (0, 1):
            pltpu.async_copy(g_hbm.at[src_i.at[b]], rows_v.at[b], sems[b])
            pltpu.async_copy(dst_hbm.at[s, b], dst_ring.at[b], dsems[b])

        def body(jj, carry):
            for b in (0, 1):
                j = jj * 2 + b
                pltpu.make_async_copy(
                    g_hbm.at[src_i.at[j]], rows_v.at[b], sems[b]).wait()
                pltpu.make_async_copy(
                    dst_hbm.at[s, j], dst_ring.at[b], dsems[b]).wait()
                pltpu.sync_copy(rows_v.at[b], acc_sp.at[dst_ring.at[b]],
                                add=True)

                @pl.when(jj < NCHUNK // 2 - 1)
                def _():
                    pltpu.async_copy(
                        g_hbm.at[src_i.at[j + 2]], rows_v.at[b], sems[b])
                    pltpu.async_copy(
                        dst_hbm.at[s, j + 2], dst_ring.at[b], dsems[b])
            return carry
        lax.fori_loop(0, NCHUNK // 2, body, 0)

    @pl.when(c == 0)
    def _():
        run(ga_hbm)

    @pl.when(c == 1)
    def _():
        run(gb_hbm)

    plsc.subcore_barrier()

    @pl.when(c == 0)
    def _():
        pltpu.sync_copy(acc_sp.at[sl], sa_hbm.at[sl])

    @pl.when(c == 1)
    def _():
        pltpu.sync_copy(acc_sp.at[sl], sb_hbm.at[sl])


_seg = pl.kernel(
    _seg_body,
    out_type=(jax.ShapeDtypeStruct((NROW, HALF), jnp.float32),
              jax.ShapeDtypeStruct((NROW, HALF), jnp.float32)),
    mesh=_MESH,
    compiler_params=_SC_PARAMS,
    scratch_types=[
        pltpu.VMEM((NCHUNK, CHUNK), jnp.int32),
        pltpu.VMEM((2, CHUNK), jnp.int32),
        pltpu.VMEM((2, CHUNK, HALF), jnp.float32),
        pltpu.SemaphoreType.DMA,
        pltpu.SemaphoreType.DMA,
        pltpu.SemaphoreType.DMA,
        pltpu.SemaphoreType.DMA,
        pltpu.VMEM_SHARED((NROW, HALF), jnp.float32),
    ],
)


def _gather_body(h0_hbm, h1_hbm, h2_hbm, ids_hbm, out_hbm, idx_v, buf_v):
    c = lax.axis_index("c")
    s = lax.axis_index("s")
    w = s * NC + c
    for which in range(3):
        pltpu.sync_copy(ids_hbm.at[which, w], idx_v)
        for l, hh in enumerate((h0_hbm, h1_hbm, h2_hbm)):
            pltpu.sync_copy(hh.at[idx_v], buf_v)
            pltpu.sync_copy(buf_v, out_hbm.at[which * 3 + l, pl.ds(w * 128, 128)])


_gather = pl.kernel(
    _gather_body,
    out_type=jax.ShapeDtypeStruct((9, BATCH, EMB), jnp.float32),
    mesh=_MESH,
    compiler_params=_SC_PARAMS,
    scratch_types=[
        pltpu.VMEM((128,), jnp.int32),
        pltpu.VMEM((128, EMB), jnp.float32),
    ],
)


def _prep_body(deg_ref, h_ref, norm_ref, ga_ref, gb_ref):
    deg = jnp.sum(deg_ref[...], axis=0)
    nrm = lax.rsqrt(jnp.maximum(deg, 1.0))
    norm_ref[...] = nrm[:, None]
    g = h_ref[...] * nrm[:, None]
    ga_ref[...] = g[:, :HALF]
    gb_ref[...] = g[:, HALF:]


_prep = pl.pallas_call(
    _prep_body,
    grid=(NROW // BLK,),
    in_specs=[
        pl.BlockSpec((NW, BLK), lambda i: (0, i)),
        pl.BlockSpec((BLK, EMB), lambda i: (i, 0)),
    ],
    out_specs=(
        pl.BlockSpec((BLK, 1), lambda i: (i, 0)),
        pl.BlockSpec((BLK, HALF), lambda i: (i, 0)),
        pl.BlockSpec((BLK, HALF), lambda i: (i, 0)),
    ),
    out_shape=(
        jax.ShapeDtypeStruct((NROW, 1), jnp.float32),
        jax.ShapeDtypeStruct((NROW, HALF), jnp.float32),
        jax.ShapeDtypeStruct((NROW, HALF), jnp.float32),
    ),
    compiler_params=pltpu.CompilerParams(dimension_semantics=("parallel",)),
)


def _layer_body(norm_ref, h_ref, sa_ref, sb_ref, w1_ref, w2_ref,
                hn_ref, ga_ref, gb_ref):
    nrm = norm_ref[...]
    h = h_ref[...]
    S = jnp.concatenate([sa_ref[...], sb_ref[...]], axis=1)
    m = jnp.dot(nrm * S + h, w1_ref[...], preferred_element_type=jnp.float32)
    m = m + jnp.dot((nrm * h) * S, w2_ref[...], preferred_element_type=jnp.float32)
    m = jnp.where(m >= 0, m, 0.2 * m)
    r = jnp.sqrt(jnp.sum(m * m, axis=1, keepdims=True))
    hn = m / jnp.maximum(r, 1e-12)
    hn_ref[...] = hn
    g = hn * nrm
    ga_ref[...] = g[:, :HALF]
    gb_ref[...] = g[:, HALF:]


_layer = pl.pallas_call(
    _layer_body,
    grid=(NROW // BLK,),
    in_specs=[
        pl.BlockSpec((BLK, 1), lambda i: (i, 0)),
        pl.BlockSpec((BLK, EMB), lambda i: (i, 0)),
        pl.BlockSpec((BLK, HALF), lambda i: (i, 0)),
        pl.BlockSpec((BLK, HALF), lambda i: (i, 0)),
        pl.BlockSpec((EMB, EMB), lambda i: (0, 0)),
        pl.BlockSpec((EMB, EMB), lambda i: (0, 0)),
    ],
    out_specs=(
        pl.BlockSpec((BLK, EMB), lambda i: (i, 0)),
        pl.BlockSpec((BLK, HALF), lambda i: (i, 0)),
        pl.BlockSpec((BLK, HALF), lambda i: (i, 0)),
    ),
    out_shape=(
        jax.ShapeDtypeStruct((NROW, EMB), jnp.float32),
        jax.ShapeDtypeStruct((NROW, HALF), jnp.float32),
        jax.ShapeDtypeStruct((NROW, HALF), jnp.float32),
    ),
    compiler_params=pltpu.CompilerParams(dimension_semantics=("parallel",)),
)


def _loss_body(x_ref, out_ref):
    i = pl.program_id(0)

    @pl.when(i == 0)
    def _():
        out_ref[...] = jnp.zeros_like(out_ref)

    pos = jnp.zeros((BB,), jnp.float32)
    neg = jnp.zeros((BB,), jnp.float32)
    reg = jnp.float32(0.0)
    for l in range(3):
        u = x_ref[l]
        p = x_ref[3 + l]
        q = x_ref[6 + l]
        pos = pos + jnp.sum(u * p, axis=1)
        neg = neg + jnp.sum(u * q, axis=1)
        reg = reg + jnp.sum(u * u) + jnp.sum(p * p) + jnp.sum(q * q)
    x = pos - neg
    ls = jnp.minimum(x, 0.0) - jnp.log1p(jnp.exp(-jnp.abs(x)))
    upd = jnp.stack([jnp.sum(ls), reg]).reshape(1, 2)
    out_ref[...] += upd


_loss = pl.pallas_call(
    _loss_body,
    grid=(BATCH // BB,),
    in_specs=[pl.BlockSpec((9, BB, EMB), lambda i: (0, i, 0))],
    out_specs=pl.BlockSpec((1, 2), lambda i: (0, 0)),
    out_shape=jax.ShapeDtypeStruct((1, 2), jnp.float32),
    compiler_params=pltpu.CompilerParams(dimension_semantics=("arbitrary",)),
)


def kernel(user_table, item_table, W1_0, W2_0, W1_1, W2_1,
           edge_index, user_id, item_id, neg_item_id):
    src = edge_index[0].astype(jnp.int32)
    dst = edge_index[1].astype(jnp.int32)
    pad = jnp.full((NE_PAD - NE,), PAD_NODE, jnp.int32)
    srcp = jnp.concatenate([src, pad])
    dstp = jnp.concatenate([dst, pad])
    src_deg = srcp.reshape(NW, EPT_DEG)
    src_r = srcp.reshape(NS, NCHUNK, CHUNK)
    dst_r = dstp.reshape(NS, NCHUNK, CHUNK)
    h0 = jnp.concatenate([user_table, item_table], axis=0)
    h0p = jnp.pad(h0, ((0, NROW - N_REAL), (0, 0)))
    zrows = jnp.zeros((ROWS_PER_TILE, HALF), jnp.float32)
    ids = jnp.stack([user_id, USER + item_id, USER + neg_item_id]) \
        .astype(jnp.int32).reshape(3, NW, 128)

    degp = _deg(src_deg)
    norm, g0a, g0b = _prep(degp, h0p)
    s0a, s0b = _seg(g0a, g0b, src_r, dst_r, zrows)
    h1p, g1a, g1b = _layer(norm, h0p, s0a, s0b, W1_0, W2_0)
    s1a, s1b = _seg(g1a, g1b, src_r, dst_r, zrows)
    h2p, _, _ = _layer(norm, h1p, s1a, s1b, W1_1, W2_1)
    gath = _gather(h0p, h1p, h2p, ids)
    acc = _loss(gath)
    return -acc[0, 0] / BATCH + LMBD * (0.5 * acc[0, 1]) / BATCH


# speedup vs baseline: 1.1542x; 1.1542x over previous
import functools

import jax
import jax.numpy as jnp
from jax import lax
from jax.experimental import pallas as pl
from jax.experimental.pallas import tpu as pltpu
from jax.experimental.pallas import tpu_sc as plsc

USER = 5000
N_REAL = 10000
NROW = 10240
EMB = 256
HALF = 128
NE = 160000
NE_PAD = 162816
PAD_NODE = 10000
NC, NS = 2, 16
NW = NC * NS
EPT_DEG = NE_PAD // NW
CHUNK = 64
NCHUNK = NE_PAD // NS // CHUNK
NBUF = 3
ROWS_PER_TILE = NROW // NS
BATCH = 4096
LMBD = 1e-05
BLK = 1280
BB = 512

_MESH = plsc.VectorSubcoreMesh(
    core_axis_name="c", subcore_axis_name="s", num_cores=NC, num_subcores=NS)


def _deg_body(src_hbm, out_hbm, src_v, hist_v):
    c = lax.axis_index("c")
    s = lax.axis_index("s")
    w = s * NC + c
    pltpu.sync_copy(src_hbm.at[w], src_v)

    def zero(i, carry):
        hist_v[pl.ds(i * 16, 16)] = jnp.zeros((16,), jnp.float32)
        return carry
    lax.fori_loop(0, NROW // 16, zero, 0)

    ones = jnp.ones((16,), jnp.float32)

    def body(i, carry):
        idx = src_v[pl.ds(i * 16, 16)]
        plsc.addupdate_scatter(hist_v, [idx], ones)
        return carry
    lax.fori_loop(0, EPT_DEG // 16, body, 0)
    pltpu.sync_copy(hist_v, out_hbm.at[w])


_SC_PARAMS = pltpu.CompilerParams(needs_layout_passes=False)

_deg = pl.kernel(
    _deg_body,
    out_type=jax.ShapeDtypeStruct((NW, NROW), jnp.float32),
    mesh=_MESH,
    compiler_params=_SC_PARAMS,
    scratch_types=[
        pltpu.VMEM((EPT_DEG,), jnp.int32),
        pltpu.VMEM((NROW,), jnp.float32),
    ],
)


def _seg_body(ga_hbm, gb_hbm, sd_hbm, z_hbm, sa_hbm, sb_hbm,
              idx_i, rows_v, sem0, sem1, sem2, acc_sp):
    c = lax.axis_index("c")
    s = lax.axis_index("s")
    pltpu.sync_copy(sd_hbm.at[s], idx_i)
    sl = pl.ds(s * ROWS_PER_TILE, ROWS_PER_TILE)
    pltpu.sync_copy(z_hbm, acc_sp.at[sl])
    plsc.subcore_barrier()

    def run(g_hbm):
        sems = (sem0, sem1, sem2)

        def src_ix(j):
            return idx_i.at[j, pl.ds(0, CHUNK)]

        def dst_ix(j):
            return idx_i.at[j, pl.ds(CHUNK, CHUNK)]

        for b in range(NBUF):
            pltpu.async_copy(g_hbm.at[src_ix(b)], rows_v.at[b], sems[b])

        def body(jj, carry):
            for b in range(NBUF):
                j = jj * NBUF + b
                pltpu.make_async_copy(
                    g_hbm.at[src_ix(j)], rows_v.at[b], sems[b]).wait()
                pltpu.sync_copy(rows_v.at[b], acc_sp.at[dst_ix(j)], add=True)

                @pl.when(jj < NCHUNK // NBUF - 1)
                def _():
                    pltpu.async_copy(
                        g_hbm.at[src_ix(j + NBUF)], rows_v.at[b], sems[b])
            return carry
        lax.fori_loop(0, NCHUNK // NBUF, body, 0)

    @pl.when(c == 0)
    def _():
        run(ga_hbm)

    @pl.when(c == 1)
    def _():
        run(gb_hbm)

    plsc.subcore_barrier()

    @pl.when(c == 0)
    def _():
        pltpu.sync_copy(acc_sp.at[sl], sa_hbm.at[sl])

    @pl.when(c == 1)
    def _():
        pltpu.sync_copy(acc_sp.at[sl], sb_hbm.at[sl])


_seg = pl.kernel(
    _seg_body,
    out_type=(jax.ShapeDtypeStruct((NROW, HALF), jnp.float32),
              jax.ShapeDtypeStruct((NROW, HALF), jnp.float32)),
    mesh=_MESH,
    compiler_params=_SC_PARAMS,
    scratch_types=[
        pltpu.VMEM((NCHUNK, 2 * CHUNK), jnp.int32),
        pltpu.VMEM((NBUF, CHUNK, HALF), jnp.float32),
        pltpu.SemaphoreType.DMA,
        pltpu.SemaphoreType.DMA,
        pltpu.SemaphoreType.DMA,
        pltpu.VMEM_SHARED((NROW, HALF), jnp.float32),
    ],
)


def _gather_body(h0_hbm, h1_hbm, h2_hbm, ids_hbm, out_hbm, idx_v, buf_v):
    c = lax.axis_index("c")
    s = lax.axis_index("s")
    w = s * NC + c
    for which in range(3):
        pltpu.sync_copy(ids_hbm.at[which, w], idx_v)
        for l, hh in enumerate((h0_hbm, h1_hbm, h2_hbm)):
            pltpu.sync_copy(hh.at[idx_v], buf_v)
            pltpu.sync_copy(buf_v, out_hbm.at[which * 3 + l, pl.ds(w * 128, 128)])


_gather = pl.kernel(
    _gather_body,
    out_type=jax.ShapeDtypeStruct((9, BATCH, EMB), jnp.float32),
    mesh=_MESH,
    compiler_params=_SC_PARAMS,
    scratch_types=[
        pltpu.VMEM((128,), jnp.int32),
        pltpu.VMEM((128, EMB), jnp.float32),
    ],
)


def _prep_body(deg_ref, h_ref, norm_ref, ga_ref, gb_ref):
    deg = jnp.sum(deg_ref[...], axis=0)
    nrm = lax.rsqrt(jnp.maximum(deg, 1.0))
    norm_ref[...] = nrm[:, None]
    g = h_ref[...] * nrm[:, None]
    ga_ref[...] = g[:, :HALF]
    gb_ref[...] = g[:, HALF:]


_prep = pl.pallas_call(
    _prep_body,
    grid=(NROW // BLK,),
    in_specs=[
        pl.BlockSpec((NW, BLK), lambda i: (0, i)),
        pl.BlockSpec((BLK, EMB), lambda i: (i, 0)),
    ],
    out_specs=(
        pl.BlockSpec((BLK, 1), lambda i: (i, 0)),
        pl.BlockSpec((BLK, HALF), lambda i: (i, 0)),
        pl.BlockSpec((BLK, HALF), lambda i: (i, 0)),
    ),
    out_shape=(
        jax.ShapeDtypeStruct((NROW, 1), jnp.float32),
        jax.ShapeDtypeStruct((NROW, HALF), jnp.float32),
        jax.ShapeDtypeStruct((NROW, HALF), jnp.float32),
    ),
    compiler_params=pltpu.CompilerParams(dimension_semantics=("parallel",)),
)


def _layer_body(norm_ref, h_ref, sa_ref, sb_ref, w1_ref, w2_ref,
                hn_ref, ga_ref, gb_ref):
    nrm = norm_ref[...]
    h = h_ref[...]
    S = jnp.concatenate([sa_ref[...], sb_ref[...]], axis=1)
    m = jnp.dot(nrm * S + h, w1_ref[...], preferred_element_type=jnp.float32)
    m = m + jnp.dot((nrm * h) * S, w2_ref[...], preferred_element_type=jnp.float32)
    m = jnp.where(m >= 0, m, 0.2 * m)
    r = jnp.sqrt(jnp.sum(m * m, axis=1, keepdims=True))
    hn = m / jnp.maximum(r, 1e-12)
    hn_ref[...] = hn
    g = hn * nrm
    ga_ref[...] = g[:, :HALF]
    gb_ref[...] = g[:, HALF:]


_layer = pl.pallas_call(
    _layer_body,
    grid=(NROW // BLK,),
    in_specs=[
        pl.BlockSpec((BLK, 1), lambda i: (i, 0)),
        pl.BlockSpec((BLK, EMB), lambda i: (i, 0)),
        pl.BlockSpec((BLK, HALF), lambda i: (i, 0)),
        pl.BlockSpec((BLK, HALF), lambda i: (i, 0)),
        pl.BlockSpec((EMB, EMB), lambda i: (0, 0)),
        pl.BlockSpec((EMB, EMB), lambda i: (0, 0)),
    ],
    out_specs=(
        pl.BlockSpec((BLK, EMB), lambda i: (i, 0)),
        pl.BlockSpec((BLK, HALF), lambda i: (i, 0)),
        pl.BlockSpec((BLK, HALF), lambda i: (i, 0)),
    ),
    out_shape=(
        jax.ShapeDtypeStruct((NROW, EMB), jnp.float32),
        jax.ShapeDtypeStruct((NROW, HALF), jnp.float32),
        jax.ShapeDtypeStruct((NROW, HALF), jnp.float32),
    ),
    compiler_params=pltpu.CompilerParams(dimension_semantics=("parallel",)),
)


def _loss_body(x_ref, out_ref):
    i = pl.program_id(0)

    @pl.when(i == 0)
    def _():
        out_ref[...] = jnp.zeros_like(out_ref)

    pos = jnp.zeros((BB,), jnp.float32)
    neg = jnp.zeros((BB,), jnp.float32)
    reg = jnp.float32(0.0)
    for l in range(3):
        u = x_ref[l]
        p = x_ref[3 + l]
        q = x_ref[6 + l]
        pos = pos + jnp.sum(u * p, axis=1)
        neg = neg + jnp.sum(u * q, axis=1)
        reg = reg + jnp.sum(u * u) + jnp.sum(p * p) + jnp.sum(q * q)
    x = pos - neg
    ls = jnp.minimum(x, 0.0) - jnp.log1p(jnp.exp(-jnp.abs(x)))
    upd = jnp.stack([jnp.sum(ls), reg]).reshape(1, 2)
    out_ref[...] += upd


_loss = pl.pallas_call(
    _loss_body,
    grid=(BATCH // BB,),
    in_specs=[pl.BlockSpec((9, BB, EMB), lambda i: (0, i, 0))],
    out_specs=pl.BlockSpec((1, 2), lambda i: (0, 0)),
    out_shape=jax.ShapeDtypeStruct((1, 2), jnp.float32),
    compiler_params=pltpu.CompilerParams(dimension_semantics=("arbitrary",)),
)


def kernel(user_table, item_table, W1_0, W2_0, W1_1, W2_1,
           edge_index, user_id, item_id, neg_item_id):
    src = edge_index[0].astype(jnp.int32)
    dst = edge_index[1].astype(jnp.int32)
    pad = jnp.full((NE_PAD - NE,), PAD_NODE, jnp.int32)
    srcp = jnp.concatenate([src, pad])
    dstp = jnp.concatenate([dst, pad])
    src_deg = srcp.reshape(NW, EPT_DEG)
    sd_r = jnp.concatenate([srcp.reshape(NS, NCHUNK, CHUNK),
                            dstp.reshape(NS, NCHUNK, CHUNK)], axis=-1)
    h0 = jnp.concatenate([user_table, item_table], axis=0)
    h0p = jnp.pad(h0, ((0, NROW - N_REAL), (0, 0)))
    zrows = jnp.zeros((ROWS_PER_TILE, HALF), jnp.float32)
    ids = jnp.stack([user_id, USER + item_id, USER + neg_item_id]) \
        .astype(jnp.int32).reshape(3, NW, 128)

    degp = _deg(src_deg)
    norm, g0a, g0b = _prep(degp, h0p)
    s0a, s0b = _seg(g0a, g0b, sd_r, zrows)
    h1p, g1a, g1b = _layer(norm, h0p, s0a, s0b, W1_0, W2_0)
    s1a, s1b = _seg(g1a, g1b, sd_r, zrows)
    h2p, _, _ = _layer(norm, h1p, s1a, s1b, W1_1, W2_1)
    gath = _gather(h0p, h1p, h2p, ids)
    acc = _loss(gath)
    return -acc[0, 0] / BATCH + LMBD * (0.5 * acc[0, 1]) / BATCH


# speedup vs baseline: 1.4062x; 1.2183x over previous
import functools

import jax
import jax.numpy as jnp
from jax import lax
from jax.experimental import pallas as pl
from jax.experimental.pallas import tpu as pltpu
from jax.experimental.pallas import tpu_sc as plsc

USER = 5000
N_REAL = 10000
NROW = 10240
EMB = 256
HALF = 128
NE = 160000
NE_PAD = 161792
PAD_NODE = 10000
NC, NS = 2, 16
NW = NC * NS
EPT_DEG = NE_PAD // NW
CHUNK = 64
NCHUNK = NE_PAD // NS // CHUNK
ROWS_PER_TILE = NROW // NS
BATCH = 4096
LMBD = 1e-05
BLK = 1280
BB = 512

_MESH = plsc.VectorSubcoreMesh(
    core_axis_name="c", subcore_axis_name="s", num_cores=NC, num_subcores=NS)


def _deg_body(src_hbm, out_hbm, src_v, hist_v):
    c = lax.axis_index("c")
    s = lax.axis_index("s")
    w = s * NC + c
    pltpu.sync_copy(src_hbm.at[w], src_v)

    def zero(i, carry):
        hist_v[pl.ds(i * 16, 16)] = jnp.zeros((16,), jnp.float32)
        return carry
    lax.fori_loop(0, NROW // 16, zero, 0)

    ones = jnp.ones((16,), jnp.float32)

    def body(i, carry):
        idx = src_v[pl.ds(i * 16, 16)]
        plsc.addupdate_scatter(hist_v, [idx], ones)
        return carry
    lax.fori_loop(0, EPT_DEG // 16, body, 0)
    pltpu.sync_copy(hist_v, out_hbm.at[w])


_SC_PARAMS = pltpu.CompilerParams(needs_layout_passes=False)

_deg = pl.kernel(
    _deg_body,
    out_type=jax.ShapeDtypeStruct((NW, NROW), jnp.float32),
    mesh=_MESH,
    compiler_params=_SC_PARAMS,
    scratch_types=[
        pltpu.VMEM((EPT_DEG,), jnp.int32),
        pltpu.VMEM((NROW,), jnp.float32),
    ],
)


def _seg_body(ga_hbm, gb_hbm, sd_hbm, z_hbm, sa_hbm, sb_hbm,
              idx_i, rows_v, sem0, sem1, acc_sp):
    c = lax.axis_index("c")
    s = lax.axis_index("s")
    pltpu.sync_copy(sd_hbm.at[s], idx_i)
    sl = pl.ds(s * ROWS_PER_TILE, ROWS_PER_TILE)
    pltpu.sync_copy(z_hbm, acc_sp.at[sl])
    plsc.subcore_barrier()

    def run(g_hbm):
        sems = (sem0, sem1)

        def src_ix(j):
            return idx_i.at[j, pl.ds(0, CHUNK)]

        def dst_ix(j):
            return idx_i.at[j, pl.ds(CHUNK, CHUNK)]

        pltpu.async_copy(g_hbm.at[src_ix(0)], rows_v.at[0], sem0)
        pltpu.async_copy(g_hbm.at[src_ix(1)], rows_v.at[1], sem1)

        def body(jj, carry):
            for b in (0, 1):
                j = jj * 2 + b
                pltpu.make_async_copy(
                    g_hbm.at[src_ix(j)], rows_v.at[b], sems[b]).wait()
                pltpu.sync_copy(rows_v.at[b], acc_sp.at[dst_ix(j)], add=True)

                @pl.when(jj < NCHUNK // 2 - 1)
                def _():
                    pltpu.async_copy(
                        g_hbm.at[src_ix(j + 2)], rows_v.at[b], sems[b])
            return carry
        lax.fori_loop(0, NCHUNK // 2, body, 0)

    @pl.when(c == 0)
    def _():
        run(ga_hbm)

    @pl.when(c == 1)
    def _():
        run(gb_hbm)

    plsc.subcore_barrier()

    @pl.when(c == 0)
    def _():
        pltpu.sync_copy(acc_sp.at[sl], sa_hbm.at[sl])

    @pl.when(c == 1)
    def _():
        pltpu.sync_copy(acc_sp.at[sl], sb_hbm.at[sl])


_seg = pl.kernel(
    _seg_body,
    out_type=(jax.ShapeDtypeStruct((NROW, HALF), jnp.float32),
              jax.ShapeDtypeStruct((NROW, HALF), jnp.float32)),
    mesh=_MESH,
    compiler_params=_SC_PARAMS,
    scratch_types=[
        pltpu.VMEM((NCHUNK, 2 * CHUNK), jnp.int32),
        pltpu.VMEM((2, CHUNK, HALF), jnp.float32),
        pltpu.SemaphoreType.DMA,
        pltpu.SemaphoreType.DMA,
        pltpu.VMEM_SHARED((NROW, HALF), jnp.float32),
    ],
)


def _gather01_body(h0_hbm, h1_hbm, ids_hbm, out_hbm, idx_v, buf_v):
    c = lax.axis_index("c")
    s = lax.axis_index("s")
    w = s * NC + c
    for which in range(3):
        pltpu.sync_copy(ids_hbm.at[which, w], idx_v)
        for l, hh in enumerate((h0_hbm, h1_hbm)):
            pltpu.sync_copy(hh.at[idx_v], buf_v)
            pltpu.sync_copy(buf_v, out_hbm.at[which * 2 + l, pl.ds(w * 128, 128)])


_gather01 = pl.kernel(
    _gather01_body,
    out_type=jax.ShapeDtypeStruct((6, BATCH, EMB), jnp.float32),
    mesh=_MESH,
    compiler_params=_SC_PARAMS,
    scratch_types=[
        pltpu.VMEM((128,), jnp.int32),
        pltpu.VMEM((128, EMB), jnp.float32),
    ],
)


def _gather2_body(h2_hbm, ids_hbm, out_hbm, idx_v, buf_v):
    c = lax.axis_index("c")
    s = lax.axis_index("s")
    w = s * NC + c
    for which in range(3):
        pltpu.sync_copy(ids_hbm.at[which, w], idx_v)
        pltpu.sync_copy(h2_hbm.at[idx_v], buf_v)
        pltpu.sync_copy(buf_v, out_hbm.at[which, pl.ds(w * 128, 128)])


_gather2 = pl.kernel(
    _gather2_body,
    out_type=jax.ShapeDtypeStruct((3, BATCH, EMB), jnp.float32),
    mesh=_MESH,
    compiler_params=_SC_PARAMS,
    scratch_types=[
        pltpu.VMEM((128,), jnp.int32),
        pltpu.VMEM((128, EMB), jnp.float32),
    ],
)


def _prep_body(deg_ref, h_ref, norm_ref, ga_ref, gb_ref):
    deg = jnp.sum(deg_ref[...], axis=0)
    nrm = lax.rsqrt(jnp.maximum(deg, 1.0))
    norm_ref[...] = nrm[:, None]
    g = h_ref[...] * nrm[:, None]
    ga_ref[...] = g[:, :HALF]
    gb_ref[...] = g[:, HALF:]


_prep = pl.pallas_call(
    _prep_body,
    grid=(NROW // BLK,),
    in_specs=[
        pl.BlockSpec((NW, BLK), lambda i: (0, i)),
        pl.BlockSpec((BLK, EMB), lambda i: (i, 0)),
    ],
    out_specs=(
        pl.BlockSpec((BLK, 1), lambda i: (i, 0)),
        pl.BlockSpec((BLK, HALF), lambda i: (i, 0)),
        pl.BlockSpec((BLK, HALF), lambda i: (i, 0)),
    ),
    out_shape=(
        jax.ShapeDtypeStruct((NROW, 1), jnp.float32),
        jax.ShapeDtypeStruct((NROW, HALF), jnp.float32),
        jax.ShapeDtypeStruct((NROW, HALF), jnp.float32),
    ),
    compiler_params=pltpu.CompilerParams(dimension_semantics=("parallel",)),
)


def _layer_body(norm_ref, h_ref, sa_ref, sb_ref, w1_ref, w2_ref,
                hn_ref, ga_ref, gb_ref):
    nrm = norm_ref[...]
    h = h_ref[...]
    S = jnp.concatenate([sa_ref[...], sb_ref[...]], axis=1)
    m = jnp.dot(nrm * S + h, w1_ref[...], preferred_element_type=jnp.float32)
    m = m + jnp.dot((nrm * h) * S, w2_ref[...], preferred_element_type=jnp.float32)
    m = jnp.where(m >= 0, m, 0.2 * m)
    r = jnp.sqrt(jnp.sum(m * m, axis=1, keepdims=True))
    hn = m / jnp.maximum(r, 1e-12)
    hn_ref[...] = hn
    g = hn * nrm
    ga_ref[...] = g[:, :HALF]
    gb_ref[...] = g[:, HALF:]


_layer = pl.pallas_call(
    _layer_body,
    grid=(NROW // BLK,),
    in_specs=[
        pl.BlockSpec((BLK, 1), lambda i: (i, 0)),
        pl.BlockSpec((BLK, EMB), lambda i: (i, 0)),
        pl.BlockSpec((BLK, HALF), lambda i: (i, 0)),
        pl.BlockSpec((BLK, HALF), lambda i: (i, 0)),
        pl.BlockSpec((EMB, EMB), lambda i: (0, 0)),
        pl.BlockSpec((EMB, EMB), lambda i: (0, 0)),
    ],
    out_specs=(
        pl.BlockSpec((BLK, EMB), lambda i: (i, 0)),
        pl.BlockSpec((BLK, HALF), lambda i: (i, 0)),
        pl.BlockSpec((BLK, HALF), lambda i: (i, 0)),
    ),
    out_shape=(
        jax.ShapeDtypeStruct((NROW, EMB), jnp.float32),
        jax.ShapeDtypeStruct((NROW, HALF), jnp.float32),
        jax.ShapeDtypeStruct((NROW, HALF), jnp.float32),
    ),
    compiler_params=pltpu.CompilerParams(dimension_semantics=("parallel",)),
)


def _loss_body(x01_ref, x2_ref, out_ref):
    i = pl.program_id(0)

    @pl.when(i == 0)
    def _():
        out_ref[...] = jnp.zeros_like(out_ref)

    pos = jnp.zeros((BB,), jnp.float32)
    neg = jnp.zeros((BB,), jnp.float32)
    reg = jnp.float32(0.0)
    for l in range(3):
        if l < 2:
            u = x01_ref[l]
            p = x01_ref[2 + l]
            q = x01_ref[4 + l]
        else:
            u = x2_ref[0]
            p = x2_ref[1]
            q = x2_ref[2]
        pos = pos + jnp.sum(u * p, axis=1)
        neg = neg + jnp.sum(u * q, axis=1)
        reg = reg + jnp.sum(u * u) + jnp.sum(p * p) + jnp.sum(q * q)
    x = pos - neg
    ls = jnp.minimum(x, 0.0) - jnp.log1p(jnp.exp(-jnp.abs(x)))
    upd = jnp.stack([jnp.sum(ls), reg]).reshape(1, 2)
    out_ref[...] += upd


_loss = pl.pallas_call(
    _loss_body,
    grid=(BATCH // BB,),
    in_specs=[pl.BlockSpec((6, BB, EMB), lambda i: (0, i, 0)),
              pl.BlockSpec((3, BB, EMB), lambda i: (0, i, 0))],
    out_specs=pl.BlockSpec((1, 2), lambda i: (0, 0)),
    out_shape=jax.ShapeDtypeStruct((1, 2), jnp.float32),
    compiler_params=pltpu.CompilerParams(dimension_semantics=("arbitrary",)),
)


def kernel(user_table, item_table, W1_0, W2_0, W1_1, W2_1,
           edge_index, user_id, item_id, neg_item_id):
    src = edge_index[0].astype(jnp.int32)
    dst = edge_index[1].astype(jnp.int32)
    pad = jnp.full((NE_PAD - NE,), PAD_NODE, jnp.int32)
    srcp = jnp.concatenate([src, pad])
    dstp = jnp.concatenate([dst, pad])
    src_deg = srcp.reshape(NW, EPT_DEG)
    sd_r = jnp.concatenate([srcp.reshape(NS, NCHUNK, CHUNK),
                            dstp.reshape(NS, NCHUNK, CHUNK)], axis=-1)
    h0 = jnp.concatenate([user_table, item_table], axis=0)
    h0p = jnp.pad(h0, ((0, NROW - N_REAL), (0, 0)))
    zrows = jnp.zeros((ROWS_PER_TILE, HALF), jnp.float32)
    ids = jnp.stack([user_id, USER + item_id, USER + neg_item_id]) \
        .astype(jnp.int32).reshape(3, NW, 128)

    degp = _deg(src_deg)
    norm, g0a, g0b = _prep(degp, h0p)
    s0a, s0b = _seg(g0a, g0b, sd_r, zrows)
    h1p, g1a, g1b = _layer(norm, h0p, s0a, s0b, W1_0, W2_0)
    s1a, s1b = _seg(g1a, g1b, sd_r, zrows)
    gath01 = _gather01(h0p, h1p, ids)
    h2p, _, _ = _layer(norm, h1p, s1a, s1b, W1_1, W2_1)
    gath2 = _gather2(h2p, ids)
    acc = _loss(gath01, gath2)
    return -acc[0, 0] / BATCH + LMBD * (0.5 * acc[0, 1]) / BATCH
